# Initial kernel scaffold; baseline (speedup 1.0000x reference)
import jax
import jax.numpy as jnp
from jax.experimental import pallas as pl


def kernel(x, edge_index, batch, W1a, b1a, W1b, b1b, W2a, b2a, W2b, b2b, W3a, b3a, W3b, b3b, Wlin, blin):
    raise NotImplementedError("write your pallas kernel here")


# speedup vs baseline: 6.7040x; 6.7040x over previous
import functools

import jax
import jax.numpy as jnp
from jax import lax
from jax.experimental import pallas as pl
from jax.experimental.pallas import tpu as pltpu
from jax.experimental.pallas import tpu_sc as plsc

F32 = jnp.float32
I32 = jnp.int32

_NC = 2
_NS = 16
_L = 16
_C = 128
_ZR = 1000
_BLK = 2000
_G = 1000


def _sc_mesh():
    return plsc.VectorSubcoreMesh(core_axis_name="c", subcore_axis_name="s")


def _fill_rows(ref, nrows, value):
    def body(i, _):
        ref[i] = jnp.full((_L,), value, F32)
        return 0
    lax.fori_loop(0, nrows, body, 0)


def _chunk_count(nrows, s):
    full, extra = divmod(nrows // _ZR, _NS)
    return full + jnp.where(s < extra, 1, 0)


def _zero_acc(acc, zbuf, s, nrows):
    def z(j, _):
        off = pl.multiple_of((j * _NS + s) * _ZR, 8)
        pltpu.sync_copy(zbuf, acc.at[pl.ds(off, _ZR)])
        return 0
    lax.fori_loop(0, _chunk_count(nrows, s), z, 0)


def _copy_out(acc, out, s, nrows):
    def z(j, _):
        off = pl.multiple_of((j * _NS + s) * _ZR, 8)
        pltpu.sync_copy(acc.at[pl.ds(off, _ZR)], out.at[pl.ds(off, _ZR)])
        return 0
    lax.fori_loop(0, _chunk_count(nrows, s), z, 0)


def _make_degree(N, E):
    full, extra = divmod(E // _C, _NC * _NS)

    @functools.partial(
        pl.kernel,
        out_type=(jax.ShapeDtypeStruct((N, _L), F32),
                  jax.ShapeDtypeStruct((N, _L), F32)),
        mesh=_sc_mesh(),
        compiler_params=pltpu.CompilerParams(use_tc_tiling_on_sc=False),
        scratch_types=[
            pltpu.VMEM((1, _C), I32),
            pltpu.VMEM((_C, _L), F32),
            pltpu.VMEM((_ZR, _L), F32),
            pltpu.VMEM_SHARED((N, _L), F32),
        ],
    )
    def deg_kernel(dst_hbm, out0, out1, idx, ones, zbuf, acc):
        c = lax.axis_index("c")
        s = lax.axis_index("s")
        w = c * _NS + s
        _fill_rows(ones, _C, 1.0)
        _fill_rows(zbuf, _ZR, 0.0)
        _zero_acc(acc, zbuf, s, N)
        plsc.subcore_barrier()

        nch = full + jnp.where(w < extra, 1, 0)

        def body(j, _):
            base = (j * (_NC * _NS) + w) * _C
            pltpu.sync_copy(dst_hbm.at[pl.ds(base, _C)], idx.at[0])
            pltpu.sync_copy(ones, acc.at[idx.at[0]], add=True)
            return 0
        lax.fori_loop(0, nch, body, 0)
        plsc.subcore_barrier()

        @pl.when(c == 0)
        def _():
            _copy_out(acc, out0, s, N)

        @pl.when(c == 1)
        def _():
            _copy_out(acc, out1, s, N)

    return deg_kernel


def _make_agg(N, E):
    full, extra = divmod(E // _C, _NS)

    @functools.partial(
        pl.kernel,
        out_type=(jax.ShapeDtypeStruct((N, _L), F32),
                  jax.ShapeDtypeStruct((N, _L), F32)),
        mesh=_sc_mesh(),
        compiler_params=pltpu.CompilerParams(use_tc_tiling_on_sc=False),
        scratch_types=[
            pltpu.VMEM((_C,), I32),
            pltpu.VMEM((1, _C), I32),
            pltpu.VMEM((_C, _L), F32),
            pltpu.VMEM((_ZR, _L), F32),
            pltpu.VMEM_SHARED((N, _L), F32),
            pltpu.SemaphoreType.DMA,
        ],
    )
    def agg_kernel(g0, g1, src_hbm, dst_hbm, out0, out1,
                   idxs, idxd, rows, zbuf, acc, sem):
        c = lax.axis_index("c")
        s = lax.axis_index("s")
        _fill_rows(zbuf, _ZR, 0.0)
        _zero_acc(acc, zbuf, s, N)
        plsc.subcore_barrier()

        nch = full + jnp.where(s < extra, 1, 0)

        def run(g_ref):
            def body(j, _):
                base = (j * _NS + s) * _C
                pltpu.sync_copy(src_hbm.at[pl.ds(base, _C)], idxs)
                pltpu.sync_copy(dst_hbm.at[pl.ds(base, _C)], idxd.at[0])
                pltpu.async_copy(g_ref.at[idxs], rows, sem).wait()
                pltpu.sync_copy(rows, acc.at[idxd.at[0]], add=True)
                return 0
            lax.fori_loop(0, nch, body, 0)

        @pl.when(c == 0)
        def _():
            run(g0)

        @pl.when(c == 1)
        def _():
            run(g1)

        plsc.subcore_barrier()

        @pl.when(c == 0)
        def _():
            _copy_out(acc, out0, s, N)

        @pl.when(c == 1)
        def _():
            _copy_out(acc, out1, s, N)

    return agg_kernel


def _make_pool(N, G):
    full_chunks, rem = divmod(N, _C)
    per, extra = divmod(full_chunks, _NS)

    @functools.partial(
        pl.kernel,
        out_type=(jax.ShapeDtypeStruct((G, _L), F32),
                  jax.ShapeDtypeStruct((G, _L), F32),
                  jax.ShapeDtypeStruct((G, _L), F32)),
        mesh=_sc_mesh(),
        compiler_params=pltpu.CompilerParams(use_tc_tiling_on_sc=False),
        scratch_types=[
            pltpu.VMEM((1, _C), I32),
            pltpu.VMEM((_C, _L), F32),
            pltpu.VMEM((_C, _L), F32),
            pltpu.VMEM((_ZR, _L), F32),
            pltpu.VMEM((1, rem), I32),
            pltpu.VMEM((rem, _L), F32),
            pltpu.VMEM_SHARED((G, _L), F32),
            pltpu.VMEM_SHARED((G, _L), F32),
        ],
    )
    def pool_kernel(h0, h1, bat_hbm, sums0, sums1, cnts,
                    idxb, rows, ones, zbuf, idxr, rowsr, acc, cacc):
        c = lax.axis_index("c")
        s = lax.axis_index("s")
        _fill_rows(ones, _C, 1.0)
        _fill_rows(zbuf, _ZR, 0.0)

        @pl.when(s == 0)
        def _():
            def z(i, _):
                pltpu.sync_copy(zbuf, acc.at[pl.ds(i * _ZR, _ZR)])
                pltpu.sync_copy(zbuf, cacc.at[pl.ds(i * _ZR, _ZR)])
                return 0
            lax.fori_loop(0, G // _ZR, z, 0)
        plsc.subcore_barrier()

        nch = per + jnp.where(s < extra, 1, 0)

        def run(h_ref):
            def body(j, _):
                base = (j * _NS + s) * _C
                pltpu.sync_copy(bat_hbm.at[pl.ds(base, _C)], idxb.at[0])
                pltpu.sync_copy(h_ref.at[pl.ds(base, _C)], rows)
                pltpu.sync_copy(rows, acc.at[idxb.at[0]], add=True)
                pltpu.sync_copy(ones, cacc.at[idxb.at[0]], add=True)
                return 0
            lax.fori_loop(0, nch, body, 0)

            @pl.when(s == _NS - 1)
            def _():
                base = full_chunks * _C
                pltpu.sync_copy(bat_hbm.at[pl.ds(base, rem)], idxr.at[0])
                pltpu.sync_copy(h_ref.at[pl.ds(base, rem)], rowsr)
                pltpu.sync_copy(rowsr, acc.at[idxr.at[0]], add=True)
                pltpu.sync_copy(ones.at[pl.ds(0, rem)],
                                cacc.at[idxr.at[0]], add=True)

        @pl.when(c == 0)
        def _():
            run(h0)

        @pl.when(c == 1)
        def _():
            run(h1)

        plsc.subcore_barrier()

        @pl.when(s == 0)
        def _():
            @pl.when(c == 0)
            def _():
                pltpu.sync_copy(acc, sums0)
                pltpu.sync_copy(cacc, cnts)

            @pl.when(c == 1)
            def _():
                pltpu.sync_copy(acc, sums1)

    return pool_kernel


def _tc_first(x, d0, d1, W):
    N, D = x.shape

    def body(x_ref, d0_ref, d1_ref, w_ref, dinv_ref, g0_ref, g1_ref):
        deg = d0_ref[:, 0:1] + d1_ref[:, 0:1] + 1.0
        dinv = lax.rsqrt(deg)
        p = jnp.dot(x_ref[...], w_ref[...], preferred_element_type=F32)
        g = dinv * p
        dinv_ref[...] = dinv
        g0_ref[...] = g[:, :_L]
        g1_ref[...] = g[:, _L:]

    return pl.pallas_call(
        body,
        grid=(N // _BLK,),
        in_specs=[
            pl.BlockSpec((_BLK, D), lambda i: (i, 0)),
            pl.BlockSpec((_BLK, _L), lambda i: (i, 0)),
            pl.BlockSpec((_BLK, _L), lambda i: (i, 0)),
            pl.BlockSpec((D, D), lambda i: (0, 0)),
        ],
        out_specs=[
            pl.BlockSpec((_BLK, 1), lambda i: (i, 0)),
            pl.BlockSpec((_BLK, _L), lambda i: (i, 0)),
            pl.BlockSpec((_BLK, _L), lambda i: (i, 0)),
        ],
        out_shape=[
            jax.ShapeDtypeStruct((N, 1), F32),
            jax.ShapeDtypeStruct((N, _L), F32),
            jax.ShapeDtypeStruct((N, _L), F32),
        ],
    )(x, d0, d1, W)


def _tc_mid(a0, a1, g0, g1, dinv, b, Wn=None, skip=None,
            emit_full=False, emit_halves=False):
    N = a0.shape[0]
    D = 2 * _L
    have_w = Wn is not None
    have_skip = skip is not None

    def body(*refs):
        refs = list(refs)
        a0_r, a1_r, g0_r, g1_r, dinv_r, b_r = refs[:6]
        refs = refs[6:]
        w_r = refs.pop(0) if have_w else None
        s_r = refs.pop(0) if have_skip else None
        outs = refs
        agg = jnp.concatenate([a0_r[...], a1_r[...]], axis=1)
        g = jnp.concatenate([g0_r[...], g1_r[...]], axis=1)
        dinv = dinv_r[...]
        h = dinv * (agg + g) + b_r[...]
        if have_skip:
            h = h + s_r[...]
        h = jnp.maximum(h, 0.0)
        k = 0
        if have_w:
            p = jnp.dot(h, w_r[...], preferred_element_type=F32)
            gn = dinv * p
            outs[k][...] = gn[:, :_L]
            outs[k + 1][...] = gn[:, _L:]
            k += 2
        if emit_full:
            outs[k][...] = h
            k += 1
        if emit_halves:
            outs[k][...] = h[:, :_L]
            outs[k + 1][...] = h[:, _L:]

    half = pl.BlockSpec((_BLK, _L), lambda i: (i, 0))
    in_specs = [half, half, half, half,
                pl.BlockSpec((_BLK, 1), lambda i: (i, 0)),
                pl.BlockSpec((1, D), lambda i: (0, 0))]
    args = [a0, a1, g0, g1, dinv, b]
    if have_w:
        in_specs.append(pl.BlockSpec((D, D), lambda i: (0, 0)))
        args.append(Wn)
    if have_skip:
        in_specs.append(pl.BlockSpec((_BLK, D), lambda i: (i, 0)))
        args.append(skip)
    out_specs, out_shape = [], []
    if have_w:
        out_specs += [half, half]
        out_shape += [jax.ShapeDtypeStruct((N, _L), F32)] * 2
    if emit_full:
        out_specs.append(pl.BlockSpec((_BLK, D), lambda i: (i, 0)))
        out_shape.append(jax.ShapeDtypeStruct((N, D), F32))
    if emit_halves:
        out_specs += [half, half]
        out_shape += [jax.ShapeDtypeStruct((N, _L), F32)] * 2
    return pl.pallas_call(
        body, grid=(N // _BLK,), in_specs=in_specs,
        out_specs=out_specs, out_shape=out_shape,
    )(*args)


def _tc_head(s0, s1, c0, Wlin, blin):
    G = s0.shape[0]
    OUT = Wlin.shape[1]

    def body(s0_r, s1_r, c_r, w_r, b_r, o_r):
        sums = jnp.concatenate([s0_r[...], s1_r[...]], axis=1)
        cnt = jnp.maximum(c_r[:, 0:1], 1.0)
        pooled = sums / cnt
        o_r[...] = jnp.dot(pooled, w_r[...], preferred_element_type=F32) + b_r[...]

    return pl.pallas_call(
        body, out_shape=jax.ShapeDtypeStruct((G, OUT), F32),
    )(s0, s1, c0, Wlin, blin)


def kernel(x, edge_index, batch, W1a, b1a, W1b, b1b, W2a, b2a, W2b, b2b,
           W3a, b3a, W3b, b3b, Wlin, blin):
    N, D = x.shape
    E = edge_index.shape[1]
    src = edge_index[0].astype(I32)
    dst = edge_index[1].astype(I32)
    bat = batch.astype(I32)
    b1a_, b1b_, b2a_, b2b_, b3a_, b3b_ = (
        v.reshape(1, D) for v in (b1a, b1b, b2a, b2b, b3a, b3b))
    blin_ = blin.reshape(1, -1)

    deg_k = _make_degree(N, E)
    agg_k = _make_agg(N, E)
    pool_k = _make_pool(N, _G)

    d0, d1 = deg_k(dst)
    dinv, g0, g1 = _tc_first(x, d0, d1, W1a)

    a0, a1 = agg_k(g0, g1, src, dst)
    g0, g1 = _tc_mid(a0, a1, g0, g1, dinv, b1a_, Wn=W1b)
    a0, a1 = agg_k(g0, g1, src, dst)
    g0, g1, u1 = _tc_mid(a0, a1, g0, g1, dinv, b1b_, Wn=W2a, skip=x,
                         emit_full=True)
    a0, a1 = agg_k(g0, g1, src, dst)
    g0, g1 = _tc_mid(a0, a1, g0, g1, dinv, b2a_, Wn=W2b)
    a0, a1 = agg_k(g0, g1, src, dst)
    g0, g1, u2 = _tc_mid(a0, a1, g0, g1, dinv, b2b_, Wn=W3a, skip=u1,
                         emit_full=True)
    a0, a1 = agg_k(g0, g1, src, dst)
    g0, g1 = _tc_mid(a0, a1, g0, g1, dinv, b3a_, Wn=W3b)
    a0, a1 = agg_k(g0, g1, src, dst)
    h0, h1 = _tc_mid(a0, a1, g0, g1, dinv, b3b_, skip=u2, emit_halves=True)

    s0, s1, c0 = pool_k(h0, h1, bat)
    return _tc_head(s0, s1, c0, Wlin, blin_)
